# baseline (speedup 1.0000x reference)
import jax
import jax.numpy as jnp
from jax import lax
from jax.experimental import pallas as pl
from jax.experimental.pallas import tpu as pltpu
from jax.experimental.pallas import tpu_sc as plsc

NC = 2
NS = 16
LANES = 16
K = 128


def _sc_mesh():
    return plsc.VectorSubcoreMesh(core_axis_name="c", subcore_axis_name="s")


def _make_sc_agg(H_sp, D, C, E, with_deg):
    RPT = H_sp // NS

    def body(h_hbm, src_hbm, dstc_hbm, cnt_hbm, z2_hbm, out_hbm, *rest):
        if with_deg:
            deg0_hbm, deg1_hbm = rest[0], rest[1]
            rest = rest[2:]
        src_v, dst_v, cnt_v, rows0, rows1, agg_sh, g0, g1 = rest[:8]
        rest = rest[8:]
        if with_deg:
            ones_v, dz_v, deg_sh = rest
        rows = [rows0, rows1]
        semg = [g0, g1]
        c = lax.axis_index("c")
        s = lax.axis_index("s")

        r0 = s * RPT
        pltpu.sync_copy(z2_hbm.at[pl.ds(r0, RPT)], agg_sh.at[pl.ds(r0, RPT)])

        pltpu.sync_copy(src_hbm.at[s], src_v)
        pltpu.sync_copy(dstc_hbm.at[c, s], dst_v)
        pltpu.sync_copy(cnt_hbm, cnt_v)
        if with_deg:
            for i in range(RPT // LANES):
                dz_v[pl.ds(i * LANES, LANES)] = jnp.zeros(
                    (LANES,), jnp.float32)
            pltpu.sync_copy(dz_v, deg_sh.at[pl.ds(r0, RPT)])
            for i in range(K // LANES):
                ones_v[pl.ds(i * LANES, LANES)] = jnp.full(
                    (LANES,), 1.0, jnp.float32)
        plsc.subcore_barrier()

        cnt0 = cnt_v[0, pl.ds(0, LANES)][0]
        m0 = (jnp.maximum(cnt0 - s, 0) + NS - 1) // NS
        n0 = (m0 + K - 1) // K
        m1 = (E - s + NS - 1) // NS
        hi1 = (m1 + K - 1) // K
        lo = jnp.where(c == 0, 0, m0 // K)
        hi = jnp.where(c == 0, n0, hi1)
        n = hi - lo

        def gfire(t, bf):
            pltpu.async_copy(h_hbm.at[src_v.at[lo + t]], rows[bf], semg[bf])

        def gwait(bf):
            pltpu.make_async_copy(h_hbm.at[src_v.at[0]], rows[bf],
                                  semg[bf]).wait()

        def scat(t, bf):
            pltpu.sync_copy(rows[bf], agg_sh.at[dst_v.at[lo + t]], add=True)

        def dscat(t):
            pltpu.sync_copy(ones_v, deg_sh.at[dst_v.at[lo + t]], add=True)

        @pl.when(n >= 1)
        def _():
            gfire(0, 0)

        @pl.when(n >= 2)
        def _():
            gfire(1, 1)

        def pair(g, carry):
            t0 = 2 * g
            t1 = t0 + 1
            gwait(0)
            scat(t0, 0)

            @pl.when(t0 + 2 < n)
            def _():
                gfire(t0 + 2, 0)

            if with_deg:
                dscat(t0)
            gwait(1)
            scat(t1, 1)

            @pl.when(t1 + 2 < n)
            def _():
                gfire(t1 + 2, 1)

            if with_deg:
                dscat(t1)
            return carry

        lax.fori_loop(0, n // 2, pair, 0)

        @pl.when(n % 2 == 1)
        def _():
            t = n - 1
            gwait(0)
            scat(t, 0)
            if with_deg:
                dscat(t)

        plsc.subcore_barrier()

        pltpu.sync_copy(agg_sh.at[pl.ds(r0, RPT)],
                        out_hbm.at[c, pl.ds(r0, RPT)])
        if with_deg:
            pltpu.sync_copy(deg_sh.at[pl.ds(r0, RPT)], dz_v)

            @pl.when(c == 0)
            def _():
                pltpu.sync_copy(dz_v, deg0_hbm.at[pl.ds(r0, RPT)])

            @pl.when(c == 1)
            def _():
                pltpu.sync_copy(dz_v, deg1_hbm.at[pl.ds(r0, RPT)])

    out_type = jax.ShapeDtypeStruct((NC, H_sp, D), jnp.float32)
    if with_deg:
        out_type = [out_type,
                    jax.ShapeDtypeStruct((H_sp,), jnp.float32),
                    jax.ShapeDtypeStruct((H_sp,), jnp.float32)]
    scratch = [
        pltpu.VMEM((C, K), jnp.int32),
        pltpu.VMEM((C, K), jnp.int32),
        pltpu.VMEM((8, 128), jnp.int32),
        pltpu.VMEM((K, D), jnp.float32),
        pltpu.VMEM((K, D), jnp.float32),
        pltpu.VMEM_SHARED((H_sp, D), jnp.float32),
        pltpu.SemaphoreType.DMA,
        pltpu.SemaphoreType.DMA,
    ]
    if with_deg:
        scratch += [
            pltpu.VMEM((K,), jnp.float32),
            pltpu.VMEM((RPT,), jnp.float32),
            pltpu.VMEM_SHARED((H_sp,), jnp.float32),
        ]

    def call(h, src_r, dstc, cnt, z2):
        kern = pl.kernel(
            body,
            out_type=out_type,
            mesh=_sc_mesh(),
            scratch_types=scratch,
        )
        return kern(h, src_r, dstc, cnt, z2)

    return call


def _tc_layer(p, degp, h, Wl, bl, Wr, relu):
    N, D = h.shape
    BM = 1000
    H = N // NC
    KB = H // BM
    grid = (N // BM,)

    def body(p_ref, deg_ref, h_ref, wl_ref, bl_ref, wr_ref, o_ref):
        agg = p_ref[0]
        deg = deg_ref[0]
        inv = 1.0 / jnp.maximum(deg, 1.0)
        z = (jnp.dot(agg, wl_ref[...], preferred_element_type=jnp.float32)
             * inv
             + bl_ref[...]
             + jnp.dot(h_ref[...], wr_ref[...],
                       preferred_element_type=jnp.float32))
        if relu:
            z = jnp.maximum(z, 0.0)
        o_ref[...] = z

    return pl.pallas_call(
        body,
        grid=grid,
        in_specs=[
            pl.BlockSpec((1, BM, D), lambda i: (i // KB, i % KB, 0)),
            pl.BlockSpec((1, BM, 1), lambda i: (i // KB, i % KB, 0)),
            pl.BlockSpec((BM, D), lambda i: (i, 0)),
            pl.BlockSpec((D, D), lambda i: (0, 0)),
            pl.BlockSpec((1, D), lambda i: (0, 0)),
            pl.BlockSpec((D, D), lambda i: (0, 0)),
        ],
        out_specs=pl.BlockSpec((BM, D), lambda i: (i, 0)),
        out_shape=jax.ShapeDtypeStruct((N, D), jnp.float32),
    )(p, degp, h, Wl, bl.reshape(1, D), Wr)


def kernel(x, edge_index, Wl0, bl0, Wr0, Wl1, bl1, Wr1, Wl2, bl2, Wr2):
    N, D = x.shape
    E = edge_index.shape[1]
    H = N // NC

    per_tile = -(-E // NS)
    C = -(-per_tile // K)
    C = -(-C // 8) * 8
    E_pad = NS * C * K
    H_sp = -(-(H + 1) // 256) * 256

    src = edge_index[0]
    dst = edge_index[1]

    own0 = dst < H
    csum = jnp.cumsum(own0.astype(jnp.int32))
    cnt0 = csum[-1]
    pos = jnp.where(own0, csum - 1,
                    cnt0 + jnp.arange(1, E + 1, dtype=jnp.int32) - csum - 1)
    src_p = jnp.zeros((E_pad,), jnp.int32).at[pos].set(src)
    dst_p = jnp.full((E_pad,), N, jnp.int32).at[pos].set(dst)

    halves = []
    for c in range(NC):
        lo = c * H
        d = dst_p - lo
        halves.append(jnp.where((d >= 0) & (d < H), d, H))
    src_r = src_p.reshape(C * K, NS).T.reshape(NS, C, K)
    dstc = jnp.stack(halves).reshape(NC, C * K, NS).transpose(
        0, 2, 1).reshape(NC, NS, C, K)
    cnt = jnp.full((8, 128), cnt0, jnp.int32)
    z2 = jnp.zeros((H_sp, D), jnp.float32)

    sc_agg0 = _make_sc_agg(H_sp, D, C, E, with_deg=True)
    sc_agg = _make_sc_agg(H_sp, D, C, E, with_deg=False)

    p0, deg0, deg1 = sc_agg0(x, src_r, dstc, cnt, z2)
    degp = jnp.stack([deg0[:H], deg1[:H]]).reshape(NC, H, 1)

    h = _tc_layer(p0[:, :H, :], degp, x, Wl0, bl0, Wr0, True)
    for Wl, bl, Wr, relu in [(Wl1, bl1, Wr1, True), (Wl2, bl2, Wr2, False)]:
        p = sc_agg(h, src_r, dstc, cnt, z2)[:, :H, :]
        h = _tc_layer(p, degp, h, Wl, bl, Wr, relu)
    return h


# speedup vs baseline: 1.2531x; 1.2531x over previous
import jax
import jax.numpy as jnp
from jax import lax
from jax.experimental import pallas as pl
from jax.experimental.pallas import tpu as pltpu
from jax.experimental.pallas import tpu_sc as plsc

NC = 2
NS = 16
LANES = 16
K = 128


def _sc_mesh():
    return plsc.VectorSubcoreMesh(core_axis_name="c", subcore_axis_name="s")


def _make_sc_agg(H_sp, D, C, with_deg):
    RPT = H_sp // NS
    CH = 1
    C2 = C // CH

    def body(h_hbm, src_hbm, dstc_hbm, z2_hbm, out_hbm, *rest):
        if with_deg:
            deg0_hbm, deg1_hbm = rest[0], rest[1]
            rest = rest[2:]
        src_v, dst_v, rows0, rows1, agg_sh, g0, g1 = rest[:7]
        rest = rest[7:]
        if with_deg:
            ones_v, dz_v, deg_sh = rest
        rows = [rows0, rows1]
        semg = [g0, g1]
        c = lax.axis_index("c")
        s = lax.axis_index("s")

        r0 = s * RPT
        pltpu.sync_copy(z2_hbm.at[pl.ds(r0, RPT)], agg_sh.at[pl.ds(r0, RPT)])

        pltpu.sync_copy(src_hbm.at[s], src_v)
        pltpu.sync_copy(dstc_hbm.at[c, s], dst_v)
        if with_deg:
            for i in range(RPT // LANES):
                dz_v[pl.ds(i * LANES, LANES)] = jnp.zeros(
                    (LANES,), jnp.float32)
            pltpu.sync_copy(dz_v, deg_sh.at[pl.ds(r0, RPT)])
            for i in range(K // LANES):
                ones_v[pl.ds(i * LANES, LANES)] = jnp.full(
                    (LANES,), 1.0, jnp.float32)
        plsc.subcore_barrier()

        def gfire(j, bf):
            pltpu.async_copy(h_hbm.at[src_v.at[j]], rows[bf], semg[bf])

        def gwait(bf):
            pltpu.make_async_copy(h_hbm.at[src_v.at[0]], rows[bf],
                                  semg[bf]).wait()

        def scat(j, bf):
            pltpu.sync_copy(rows[bf], agg_sh.at[dst_v.at[j]], add=True)

        def dscat(j):
            pltpu.sync_copy(ones_v, deg_sh.at[dst_v.at[j]], add=True)

        gfire(0, 0)
        gfire(1, 1)

        def step(g, carry):
            j0 = 2 * g
            j1 = j0 + 1
            gwait(0)
            scat(j0, 0)
            gfire((j0 + 2) % C2, 0)
            if with_deg:
                dscat(j0)
            gwait(1)
            scat(j1, 1)
            gfire((j1 + 2) % C2, 1)
            if with_deg:
                dscat(j1)
            return carry

        lax.fori_loop(0, C2 // 2, step, 0)
        gwait(0)
        gwait(1)
        plsc.subcore_barrier()

        pltpu.sync_copy(agg_sh.at[pl.ds(r0, RPT)],
                        out_hbm.at[c, pl.ds(r0, RPT)])
        if with_deg:
            pltpu.sync_copy(deg_sh.at[pl.ds(r0, RPT)], dz_v)

            @pl.when(c == 0)
            def _():
                pltpu.sync_copy(dz_v, deg0_hbm.at[pl.ds(r0, RPT)])

            @pl.when(c == 1)
            def _():
                pltpu.sync_copy(dz_v, deg1_hbm.at[pl.ds(r0, RPT)])

    out_type = jax.ShapeDtypeStruct((NC, H_sp, D), jnp.float32)
    if with_deg:
        out_type = [out_type,
                    jax.ShapeDtypeStruct((H_sp,), jnp.float32),
                    jax.ShapeDtypeStruct((H_sp,), jnp.float32)]
    scratch = [
        pltpu.VMEM((C, K), jnp.int32),
        pltpu.VMEM((C, K), jnp.int32),
        pltpu.VMEM((CH * K, D), jnp.float32),
        pltpu.VMEM((CH * K, D), jnp.float32),
        pltpu.VMEM_SHARED((H_sp, D), jnp.float32),
        pltpu.SemaphoreType.DMA,
        pltpu.SemaphoreType.DMA,
    ]
    if with_deg:
        scratch += [
            pltpu.VMEM((K,), jnp.float32),
            pltpu.VMEM((RPT,), jnp.float32),
            pltpu.VMEM_SHARED((H_sp,), jnp.float32),
        ]

    def call(h, src_r, dstc, z2):
        kern = pl.kernel(
            body,
            out_type=out_type,
            mesh=_sc_mesh(),
            scratch_types=scratch,
        )
        return kern(h, src_r, dstc, z2)

    return call


def _tc_layer(p, degp, h, Wl, bl, Wr, relu):
    N, D = h.shape
    BM = 1000
    H = N // NC
    KB = H // BM
    grid = (N // BM,)

    def body(p_ref, deg_ref, h_ref, wl_ref, bl_ref, wr_ref, o_ref):
        agg = p_ref[0]
        deg = deg_ref[0]
        inv = 1.0 / jnp.maximum(deg, 1.0)
        z = (jnp.dot(agg, wl_ref[...], preferred_element_type=jnp.float32)
             * inv
             + bl_ref[...]
             + jnp.dot(h_ref[...], wr_ref[...],
                       preferred_element_type=jnp.float32))
        if relu:
            z = jnp.maximum(z, 0.0)
        o_ref[...] = z

    return pl.pallas_call(
        body,
        grid=grid,
        in_specs=[
            pl.BlockSpec((1, BM, D), lambda i: (i // KB, i % KB, 0)),
            pl.BlockSpec((1, BM, 1), lambda i: (i // KB, i % KB, 0)),
            pl.BlockSpec((BM, D), lambda i: (i, 0)),
            pl.BlockSpec((D, D), lambda i: (0, 0)),
            pl.BlockSpec((1, D), lambda i: (0, 0)),
            pl.BlockSpec((D, D), lambda i: (0, 0)),
        ],
        out_specs=pl.BlockSpec((BM, D), lambda i: (i, 0)),
        out_shape=jax.ShapeDtypeStruct((N, D), jnp.float32),
    )(p, degp, h, Wl, bl.reshape(1, D), Wr)


def kernel(x, edge_index, Wl0, bl0, Wr0, Wl1, bl1, Wr1, Wl2, bl2, Wr2):
    N, D = x.shape
    E = edge_index.shape[1]
    H = N // NC

    per_tile = -(-E // NS)
    C = -(-per_tile // K)
    C = -(-C // 8) * 8
    E_pad = NS * C * K
    H_sp = -(-(H + 64) // 256) * 256

    src = edge_index[0]
    dst = edge_index[1]
    pad = E_pad - E
    src_r = jnp.concatenate(
        [src, jnp.zeros((pad,), jnp.int32)]).reshape(NS, C, K)
    dst_p = jnp.concatenate([dst, jnp.full((pad,), N, jnp.int32)])
    junk = H + (jnp.arange(E_pad, dtype=jnp.int32) & 63)
    halves = []
    for c in range(NC):
        lo = c * H
        d = dst_p - lo
        halves.append(jnp.where((d >= 0) & (d < H), d, junk))
    dstc = jnp.stack(halves).reshape(NC, NS, C, K)
    z2 = jnp.zeros((H_sp, D), jnp.float32)

    sc_agg0 = _make_sc_agg(H_sp, D, C, with_deg=True)
    sc_agg = _make_sc_agg(H_sp, D, C, with_deg=False)

    p0, deg0, deg1 = sc_agg0(x, src_r, dstc, z2)
    degp = jnp.stack([deg0[:H], deg1[:H]]).reshape(NC, H, 1)

    h = _tc_layer(p0[:, :H, :], degp, x, Wl0, bl0, Wr0, True)
    for Wl, bl, Wr, relu in [(Wl1, bl1, Wr1, True), (Wl2, bl2, Wr2, False)]:
        p = sc_agg(h, src_r, dstc, z2)[:, :H, :]
        h = _tc_layer(p, degp, h, Wl, bl, Wr, relu)
    return h


# speedup vs baseline: 1.2599x; 1.0055x over previous
import jax
import jax.numpy as jnp
from jax import lax
from jax.experimental import pallas as pl
from jax.experimental.pallas import tpu as pltpu
from jax.experimental.pallas import tpu_sc as plsc

NC = 2
NS = 16
LANES = 16
K = 128


def _sc_mesh():
    return plsc.VectorSubcoreMesh(core_axis_name="c", subcore_axis_name="s")


def _make_sc_agg(H_sp, D, C, with_deg):
    RPT = H_sp // NS
    CH = 1
    C2 = C // CH

    def body(h_hbm, src_hbm, dstc_hbm, z2_hbm, out_hbm, *rest):
        if with_deg:
            deg0_hbm, deg1_hbm = rest[0], rest[1]
            rest = rest[2:]
        src_v, dst_v, rows0, rows1, agg_sh, g0, g1 = rest[:7]
        rest = rest[7:]
        if with_deg:
            ones_v, dz_v, deg_sh = rest
        rows = [rows0, rows1]
        semg = [g0, g1]
        c = lax.axis_index("c")
        s = lax.axis_index("s")

        r0 = s * RPT
        pltpu.sync_copy(z2_hbm.at[pl.ds(r0, RPT)], agg_sh.at[pl.ds(r0, RPT)])

        pltpu.sync_copy(src_hbm.at[s], src_v)
        pltpu.sync_copy(dstc_hbm.at[c, s], dst_v)
        if with_deg:
            for i in range(RPT // LANES):
                dz_v[pl.ds(i * LANES, LANES)] = jnp.zeros(
                    (LANES,), jnp.float32)
            pltpu.sync_copy(dz_v, deg_sh.at[pl.ds(r0, RPT)])
            for i in range(K // LANES):
                ones_v[pl.ds(i * LANES, LANES)] = jnp.full(
                    (LANES,), 1.0, jnp.float32)
        plsc.subcore_barrier()

        def gfire(j, bf):
            pltpu.async_copy(h_hbm.at[src_v.at[j]], rows[bf], semg[bf])

        def gwait(bf):
            pltpu.make_async_copy(h_hbm.at[src_v.at[0]], rows[bf],
                                  semg[bf]).wait()

        def scat(j, bf):
            pltpu.sync_copy(rows[bf], agg_sh.at[dst_v.at[j]], add=True)

        def dscat(j):
            pltpu.sync_copy(ones_v, deg_sh.at[dst_v.at[j]], add=True)

        gfire(0, 0)
        gfire(1, 1)

        def step(g, carry):
            j0 = 2 * g
            j1 = j0 + 1
            gwait(0)
            scat(j0, 0)
            gfire((j0 + 2) % C2, 0)
            if with_deg:
                dscat(j0)
            gwait(1)
            scat(j1, 1)
            gfire((j1 + 2) % C2, 1)
            if with_deg:
                dscat(j1)
            return carry

        lax.fori_loop(0, C2 // 2, step, 0)
        gwait(0)
        gwait(1)
        plsc.subcore_barrier()

        pltpu.sync_copy(agg_sh.at[pl.ds(r0, RPT)],
                        out_hbm.at[c, pl.ds(r0, RPT)])
        if with_deg:
            pltpu.sync_copy(deg_sh.at[pl.ds(r0, RPT)], dz_v)

            @pl.when(c == 0)
            def _():
                pltpu.sync_copy(dz_v, deg0_hbm.at[pl.ds(r0, RPT)])

            @pl.when(c == 1)
            def _():
                pltpu.sync_copy(dz_v, deg1_hbm.at[pl.ds(r0, RPT)])

    out_type = jax.ShapeDtypeStruct((NC, H_sp, D), jnp.float32)
    if with_deg:
        out_type = [out_type,
                    jax.ShapeDtypeStruct((H_sp,), jnp.float32),
                    jax.ShapeDtypeStruct((H_sp,), jnp.float32)]
    scratch = [
        pltpu.VMEM((C, K), jnp.int32),
        pltpu.VMEM((C, K), jnp.int32),
        pltpu.VMEM((CH * K, D), jnp.float32),
        pltpu.VMEM((CH * K, D), jnp.float32),
        pltpu.VMEM_SHARED((H_sp, D), jnp.float32),
        pltpu.SemaphoreType.DMA,
        pltpu.SemaphoreType.DMA,
    ]
    if with_deg:
        scratch += [
            pltpu.VMEM((K,), jnp.float32),
            pltpu.VMEM((RPT,), jnp.float32),
            pltpu.VMEM_SHARED((H_sp,), jnp.float32),
        ]

    def call(h, src_r, dstc, z2):
        kern = pl.kernel(
            body,
            out_type=out_type,
            mesh=_sc_mesh(),
            scratch_types=scratch,
        )
        return kern(h, src_r, dstc, z2)

    return call


def _tc_layer(p, degp, h, Wl, bl, Wr, relu):
    N, D = h.shape
    BM = 1000
    H = N // NC
    KB = H // BM
    grid = (N // BM,)

    def body(p_ref, deg_ref, h_ref, wl_ref, bl_ref, wr_ref, o_ref):
        agg = p_ref[0]
        deg = deg_ref[0]
        inv = 1.0 / jnp.maximum(deg, 1.0)
        z = (jnp.dot(agg, wl_ref[...], preferred_element_type=jnp.float32)
             * inv
             + bl_ref[...]
             + jnp.dot(h_ref[...], wr_ref[...],
                       preferred_element_type=jnp.float32))
        if relu:
            z = jnp.maximum(z, 0.0)
        o_ref[...] = z

    return pl.pallas_call(
        body,
        grid=grid,
        in_specs=[
            pl.BlockSpec((1, BM, D), lambda i: (i // KB, i % KB, 0)),
            pl.BlockSpec((1, BM, 1), lambda i: (i // KB, i % KB, 0)),
            pl.BlockSpec((BM, D), lambda i: (i, 0)),
            pl.BlockSpec((D, D), lambda i: (0, 0)),
            pl.BlockSpec((1, D), lambda i: (0, 0)),
            pl.BlockSpec((D, D), lambda i: (0, 0)),
        ],
        out_specs=pl.BlockSpec((BM, D), lambda i: (i, 0)),
        out_shape=jax.ShapeDtypeStruct((N, D), jnp.float32),
    )(p, degp, h, Wl, bl.reshape(1, D), Wr)


def kernel(x, edge_index, Wl0, bl0, Wr0, Wl1, bl1, Wr1, Wl2, bl2, Wr2):
    N, D = x.shape
    E = edge_index.shape[1]
    H = N // NC

    per_tile = -(-E // NS)
    C = -(-per_tile // K)
    C = -(-C // 8) * 8
    E_pad = NS * C * K
    H_sp = -(-(H + 512) // 256) * 256

    src = edge_index[0]
    dst = edge_index[1]
    pad = E_pad - E
    src_r = jnp.concatenate(
        [src, jnp.zeros((pad,), jnp.int32)]).reshape(NS, C, K)
    dst_p = jnp.concatenate([dst, jnp.full((pad,), N, jnp.int32)])
    junk = H + (jnp.arange(E_pad, dtype=jnp.int32) & 511)
    halves = []
    for c in range(NC):
        lo = c * H
        d = dst_p - lo
        halves.append(jnp.where((d >= 0) & (d < H), d, junk))
    dstc = jnp.stack(halves).reshape(NC, NS, C, K)
    z2 = jnp.zeros((H_sp, D), jnp.float32)

    sc_agg0 = _make_sc_agg(H_sp, D, C, with_deg=True)
    sc_agg = _make_sc_agg(H_sp, D, C, with_deg=False)

    p0, deg0, deg1 = sc_agg0(x, src_r, dstc, z2)
    degp = jnp.stack([deg0[:H], deg1[:H]]).reshape(NC, H, 1)

    h = _tc_layer(p0[:, :H, :], degp, x, Wl0, bl0, Wr0, True)
    for Wl, bl, Wr, relu in [(Wl1, bl1, Wr1, True), (Wl2, bl2, Wr2, False)]:
        p = sc_agg(h, src_r, dstc, z2)[:, :H, :]
        h = _tc_layer(p, degp, h, Wl, bl, Wr, relu)
    return h
